# baseline (device time: 245992 ns/iter reference)
import functools

import jax
import jax.numpy as jnp
from jax import lax
from jax.experimental import pallas as pl
from jax.experimental.pallas import tpu as pltpu

N_DEV = 32
N_EXP = 64
CAPACITY = 204

_DID = getattr(pl, "DeviceIdType", None) or pltpu.DeviceIdType
_CP = getattr(pltpu, "CompilerParams", None) or pltpu.TPUCompilerParams


def kernel(x, router_W, route_idx, expert_W):
    del router_W
    n_tok, d_model = x.shape
    e_per, _, d_out = expert_W.shape

    x_bf = x.astype(jnp.bfloat16)
    w_bf = expert_W.astype(jnp.bfloat16)

    def body(x_ref, idx_ref, w_ref, out_ref,
             w_all, c_all, w_send, w_recv, c_send, c_recv):
        my = lax.axis_index("i")
        left = (my - 1) % N_DEV
        right = (my + 1) % N_DEV

        barrier = pltpu.get_barrier_semaphore()
        for nbr in (left, right):
            pl.semaphore_signal(barrier, inc=1, device_id=(nbr,),
                                device_id_type=_DID.MESH)
        pl.semaphore_wait(barrier, 2)

        e = idx_ref[:, :]
        onehot = (e == lax.broadcasted_iota(
            jnp.int32, (n_tok, N_EXP), 1)).astype(jnp.float32)
        counts_me = jnp.sum(onehot, axis=0, keepdims=True)

        row = lax.broadcasted_iota(jnp.int32, (n_tok, n_tok), 0)
        col = lax.broadcasted_iota(jnp.int32, (n_tok, n_tok), 1)
        tri = (row > col).astype(jnp.float32)
        lr_full = jnp.dot(tri, onehot,
                          preferred_element_type=jnp.float32)
        local_rank = jnp.sum(lr_full * onehot, axis=1,
                             keepdims=True)

        w_all[0] = w_ref[:, :, :]
        c_all[0] = counts_me

        out_ref[:, :] = jnp.zeros((n_tok, d_out), jnp.float32)
        x_v = x_ref[:, :]

        def compute(slot):
            origin = (my - slot) % N_DEV
            for s in range(e_per):
                eid = origin * e_per + s
                m = (e == eid).astype(jnp.bfloat16)
                out_ref[:, :] += jnp.dot(
                    x_v * m, w_all[slot, s],
                    preferred_element_type=jnp.float32)

        sent = []
        for h in range(N_DEV - 1):
            w_rdma = pltpu.make_async_remote_copy(
                src_ref=w_all.at[h], dst_ref=w_all.at[h + 1],
                send_sem=w_send.at[h], recv_sem=w_recv.at[h],
                device_id=(right,), device_id_type=_DID.MESH)
            c_rdma = pltpu.make_async_remote_copy(
                src_ref=c_all.at[h], dst_ref=c_all.at[h + 1],
                send_sem=c_send.at[h], recv_sem=c_recv.at[h],
                device_id=(right,), device_id_type=_DID.MESH)
            w_rdma.start()
            c_rdma.start()
            compute(h)
            w_rdma.wait_recv()
            c_rdma.wait_recv()
            sent.append((w_rdma, c_rdma))
        compute(N_DEV - 1)

        C = c_all[:, 0, :]
        kidx = lax.broadcasted_iota(jnp.int32, (N_DEV, 1), 0)
        dev_mask = ((kidx >= 1) & (kidx <= my)).astype(jnp.float32)
        prefix = jnp.sum(C * dev_mask, axis=0, keepdims=True)
        prior = jnp.sum(onehot * prefix, axis=1, keepdims=True)
        keep = ((prior + local_rank) < CAPACITY).astype(jnp.float32)
        out_ref[:, :] = out_ref[:, :] * keep

        for w_rdma, c_rdma in sent:
            w_rdma.wait_send()
            c_rdma.wait_send()

        @functools.partial(pl.run_scoped,
                           sem2=pltpu.SemaphoreType.REGULAR)
        def _(sem2):
            for nbr in (left, right):
                pl.semaphore_signal(sem2, inc=1, device_id=(nbr,),
                                    device_id_type=_DID.MESH)
            pl.semaphore_wait(sem2, 2)

    return pl.pallas_call(
        body,
        out_shape=jax.ShapeDtypeStruct((n_tok, d_out), jnp.float32),
        in_specs=[
            pl.BlockSpec(memory_space=pltpu.VMEM),
            pl.BlockSpec(memory_space=pltpu.VMEM),
            pl.BlockSpec(memory_space=pltpu.VMEM),
        ],
        out_specs=pl.BlockSpec(memory_space=pltpu.VMEM),
        scratch_shapes=[
            pltpu.VMEM((N_DEV, e_per, d_model, d_out), jnp.bfloat16),
            pltpu.VMEM((N_DEV, 1, N_EXP), jnp.float32),
            pltpu.SemaphoreType.DMA((N_DEV - 1,)),
            pltpu.SemaphoreType.DMA((N_DEV - 1,)),
            pltpu.SemaphoreType.DMA((N_DEV - 1,)),
            pltpu.SemaphoreType.DMA((N_DEV - 1,)),
        ],
        compiler_params=_CP(collective_id=0),
    )(x_bf, route_idx, w_bf)


# device time: 219997 ns/iter; 1.1182x vs baseline; 1.1182x over previous
import functools

import jax
import jax.numpy as jnp
from jax import lax
from jax.experimental import pallas as pl
from jax.experimental.pallas import tpu as pltpu

N_DEV = 32
N_EXP = 64
CAPACITY = 204
CW_HOPS = N_DEV // 2
CCW_HOPS = N_DEV // 2 - 1

_DID = getattr(pl, "DeviceIdType", None) or pltpu.DeviceIdType
_CP = getattr(pltpu, "CompilerParams", None) or pltpu.TPUCompilerParams


def kernel(x, router_W, route_idx, expert_W):
    del router_W
    n_tok, d_model = x.shape
    e_per, _, d_out = expert_W.shape

    x_bf = x.astype(jnp.bfloat16)
    w_bf = expert_W.astype(jnp.bfloat16)

    def body(x_ref, idx_ref, w_ref, out_ref,
             w_cw, w_ccw, c_cw, c_ccw,
             ws_cw, wr_cw, ws_ccw, wr_ccw,
             cs_cw, cr_cw, cs_ccw, cr_ccw):
        my = lax.axis_index("i")
        left = (my - 1) % N_DEV
        right = (my + 1) % N_DEV

        barrier = pltpu.get_barrier_semaphore()
        for nbr in (left, right):
            pl.semaphore_signal(barrier, inc=1, device_id=(nbr,),
                                device_id_type=_DID.MESH)
        pl.semaphore_wait(barrier, 2)

        e = idx_ref[:, :]
        onehot = (e == lax.broadcasted_iota(
            jnp.int32, (n_tok, N_EXP), 1)).astype(jnp.float32)
        counts_me = jnp.sum(onehot, axis=0, keepdims=True)

        row = lax.broadcasted_iota(jnp.int32, (n_tok, n_tok), 0)
        col = lax.broadcasted_iota(jnp.int32, (n_tok, n_tok), 1)
        tri = (row > col).astype(jnp.float32)
        lr_full = jnp.dot(tri, onehot,
                          preferred_element_type=jnp.float32)
        local_rank = jnp.sum(lr_full * onehot, axis=1,
                             keepdims=True)

        w_cw[0] = w_ref[:, :, :]
        w_ccw[0] = w_ref[:, :, :]
        c_cw[0] = counts_me
        c_ccw[0] = counts_me

        out_ref[:, :] = jnp.zeros((n_tok, d_out), jnp.float32)
        x_v = x_ref[:, :]

        def compute(w_buf, slot, origin):
            for s in range(e_per):
                eid = origin * e_per + s
                m = (e == eid).astype(jnp.bfloat16)
                out_ref[:, :] += jnp.dot(
                    x_v * m, w_buf[slot, s],
                    preferred_element_type=jnp.float32)

        sent = []
        for h in range(CW_HOPS):
            w_rdma = pltpu.make_async_remote_copy(
                src_ref=w_cw.at[h], dst_ref=w_cw.at[h + 1],
                send_sem=ws_cw.at[h], recv_sem=wr_cw.at[h],
                device_id=(right,), device_id_type=_DID.MESH)
            c_rdma = pltpu.make_async_remote_copy(
                src_ref=c_cw.at[h], dst_ref=c_cw.at[h + 1],
                send_sem=cs_cw.at[h], recv_sem=cr_cw.at[h],
                device_id=(right,), device_id_type=_DID.MESH)
            w_rdma.start()
            c_rdma.start()
            sent.append((w_rdma, c_rdma))
            if h < CCW_HOPS:
                w_rdma2 = pltpu.make_async_remote_copy(
                    src_ref=w_ccw.at[h], dst_ref=w_ccw.at[h + 1],
                    send_sem=ws_ccw.at[h], recv_sem=wr_ccw.at[h],
                    device_id=(left,), device_id_type=_DID.MESH)
                c_rdma2 = pltpu.make_async_remote_copy(
                    src_ref=c_ccw.at[h], dst_ref=c_ccw.at[h + 1],
                    send_sem=cs_ccw.at[h], recv_sem=cr_ccw.at[h],
                    device_id=(left,), device_id_type=_DID.MESH)
                w_rdma2.start()
                c_rdma2.start()
                sent.append((w_rdma2, c_rdma2))

            compute(w_cw, h, (my - h) % N_DEV)
            if 0 < h:
                compute(w_ccw, h, (my + h) % N_DEV)

            w_rdma.wait_recv()
            c_rdma.wait_recv()
            if h < CCW_HOPS:
                w_rdma2.wait_recv()
                c_rdma2.wait_recv()
        compute(w_cw, CW_HOPS, (my - CW_HOPS) % N_DEV)

        C1 = c_cw[:, 0, :]
        k1 = lax.broadcasted_iota(jnp.int32, (CW_HOPS + 1, 1), 0)
        m1 = ((k1 >= 1) & (k1 <= my)).astype(jnp.float32)
        C2 = c_ccw[:, 0, :]
        k2 = lax.broadcasted_iota(jnp.int32, (CCW_HOPS + 1, 1), 0)
        m2 = ((k2 >= 1) & (k2 + my >= N_DEV)).astype(jnp.float32)
        prefix = (jnp.sum(C1 * m1, axis=0, keepdims=True)
                  + jnp.sum(C2 * m2, axis=0, keepdims=True))
        prior = jnp.sum(onehot * prefix, axis=1, keepdims=True)
        keep = ((prior + local_rank) < CAPACITY).astype(jnp.float32)
        out_ref[:, :] = out_ref[:, :] * keep

        for w_rdma, c_rdma in sent:
            w_rdma.wait_send()
            c_rdma.wait_send()

        @functools.partial(pl.run_scoped,
                           sem2=pltpu.SemaphoreType.REGULAR)
        def _(sem2):
            for nbr in (left, right):
                pl.semaphore_signal(sem2, inc=1, device_id=(nbr,),
                                    device_id_type=_DID.MESH)
            pl.semaphore_wait(sem2, 2)

    return pl.pallas_call(
        body,
        out_shape=jax.ShapeDtypeStruct((n_tok, d_out), jnp.float32),
        in_specs=[
            pl.BlockSpec(memory_space=pltpu.VMEM),
            pl.BlockSpec(memory_space=pltpu.VMEM),
            pl.BlockSpec(memory_space=pltpu.VMEM),
        ],
        out_specs=pl.BlockSpec(memory_space=pltpu.VMEM),
        scratch_shapes=[
            pltpu.VMEM((CW_HOPS + 1, e_per, d_model, d_out), jnp.bfloat16),
            pltpu.VMEM((CCW_HOPS + 1, e_per, d_model, d_out), jnp.bfloat16),
            pltpu.VMEM((CW_HOPS + 1, 1, N_EXP), jnp.float32),
            pltpu.VMEM((CCW_HOPS + 1, 1, N_EXP), jnp.float32),
            pltpu.SemaphoreType.DMA((CW_HOPS,)),
            pltpu.SemaphoreType.DMA((CW_HOPS,)),
            pltpu.SemaphoreType.DMA((CCW_HOPS,)),
            pltpu.SemaphoreType.DMA((CCW_HOPS,)),
            pltpu.SemaphoreType.DMA((CW_HOPS,)),
            pltpu.SemaphoreType.DMA((CW_HOPS,)),
            pltpu.SemaphoreType.DMA((CCW_HOPS,)),
            pltpu.SemaphoreType.DMA((CCW_HOPS,)),
        ],
        compiler_params=_CP(collective_id=0),
    )(x_bf, route_idx, w_bf)
